# baseline (speedup 1.0000x reference)
import jax
import jax.numpy as jnp
from jax.experimental import pallas as pl

P = 512
EMB = 32
H = 64
B = 16
E1 = EMB + 1


def _cell_kernel(aux_ref, ht_ref, ct_ref, wfront_ref, bfront_ref,
                 wih_ref, whh_ref, hout_ref, cout_ref):
    wfront = wfront_ref[...]
    bfront = bfront_ref[...]
    wih = wih_ref[...]
    whh = whh_ref[...]
    for k in range(B):
        ht = ht_ref[k]
        ct = ct_ref[k]
        front = jnp.maximum(
            jnp.dot(wfront, aux_ref[:, k, :],
                    preferred_element_type=jnp.float32) + bfront,
            0.0)
        emb = front[:E1, :]
        m = front[E1:, :]
        gates = (jnp.dot(wih, emb, preferred_element_type=jnp.float32) +
                 jnp.dot(whh, ht, preferred_element_type=jnp.float32))
        t = jnp.tanh(gates)
        sig = 0.5 + 0.5 * t[0:3 * H, :]
        i_g = sig[0 * H:1 * H, :]
        f_g = sig[1 * H:2 * H, :]
        o_g = sig[2 * H:3 * H, :]
        g_g = t[3 * H:4 * H, :]
        c_new = f_g * ct + i_g * g_g
        h_new = o_g * jnp.tanh(c_new)
        hout_ref[k] = ht + m * (h_new - ht)
        cout_ref[k] = ct + m * (c_new - ct)


def _prep_gate_weights(W_ih, b_ih, W_hh, b_hh):
    def reorder(w):
        g4 = w.reshape(4, H, -1)
        return jnp.concatenate(
            [0.5 * g4[0], 0.5 * g4[1], 0.5 * g4[3], g4[2]], axis=0)

    wih = reorder(jnp.concatenate([W_ih, (b_ih + b_hh)[:, None]], axis=1))
    whh = reorder(W_hh)
    return wih, whh


def kernel(corr_index, rela_ht, rela_ct, nei_index, W_emb, b_emb, W_ih, b_ih,
           W_hh, b_hh):
    htT = rela_ht.transpose(0, 2, 1)
    ctT = rela_ct.transpose(0, 2, 1)
    aux = jnp.concatenate([
        corr_index.transpose(2, 0, 1),
        nei_index.astype(jnp.float32)[None, :, :],
    ], axis=0)
    wfront = jnp.zeros((E1 + H, 3), dtype=jnp.float32)
    wfront = wfront.at[:EMB, 0:2].set(W_emb)
    wfront = wfront.at[E1:, 2].set(1.0)
    bfront = jnp.concatenate(
        [b_emb, jnp.ones((1,), jnp.float32), jnp.zeros((H,), jnp.float32)])
    bfront = bfront.reshape(E1 + H, 1)
    wih, whh = _prep_gate_weights(W_ih, b_ih, W_hh, b_hh)

    ht_out, ct_out = pl.pallas_call(
        _cell_kernel,
        grid=(P // B,),
        in_specs=[
            pl.BlockSpec((3, B, P), lambda i: (0, i, 0)),
            pl.BlockSpec((B, H, P), lambda i: (i, 0, 0)),
            pl.BlockSpec((B, H, P), lambda i: (i, 0, 0)),
            pl.BlockSpec((E1 + H, 3), lambda i: (0, 0)),
            pl.BlockSpec((E1 + H, 1), lambda i: (0, 0)),
            pl.BlockSpec((4 * H, E1), lambda i: (0, 0)),
            pl.BlockSpec((4 * H, H), lambda i: (0, 0)),
        ],
        out_specs=[
            pl.BlockSpec((B, H, P), lambda i: (i, 0, 0)),
            pl.BlockSpec((B, H, P), lambda i: (i, 0, 0)),
        ],
        out_shape=[jax.ShapeDtypeStruct((P, H, P), jnp.float32)] * 2,
    )(aux, htT, ctT, wfront, bfront, wih, whh)
    return ht_out.transpose(0, 2, 1), ct_out.transpose(0, 2, 1)


# speedup vs baseline: 1.1243x; 1.1243x over previous
import jax
import jax.numpy as jnp
from jax.experimental import pallas as pl

P = 512
EMB = 32
H = 64
B = 16


def _cell_kernel(aux_ref, ht_ref, ct_ref, wfront_ref, bfront_ref,
                 wih_ref, whh_ref, b_ref, hout_ref, cout_ref):
    wfront = wfront_ref[...]
    bfront = bfront_ref[...]
    wih = wih_ref[...]
    whh = whh_ref[...]
    b = b_ref[...]
    for k in range(B):
        ht = ht_ref[k]
        ct = ct_ref[k]
        front = jnp.maximum(
            jnp.dot(wfront, aux_ref[:, k, :],
                    preferred_element_type=jnp.float32) + bfront,
            0.0)
        emb = front[:EMB, :]
        m = front[EMB:, :]
        gates = (jnp.dot(wih, emb, preferred_element_type=jnp.float32) +
                 jnp.dot(whh, ht, preferred_element_type=jnp.float32) + b)
        t = jnp.tanh(gates)
        sig = 0.5 + 0.5 * t[0:3 * H, :]
        i_g = sig[0 * H:1 * H, :]
        f_g = sig[1 * H:2 * H, :]
        o_g = sig[2 * H:3 * H, :]
        g_g = t[3 * H:4 * H, :]
        c_new = f_g * ct + i_g * g_g
        h_new = o_g * jnp.tanh(c_new)
        hout_ref[k] = ht + m * (h_new - ht)
        cout_ref[k] = ct + m * (c_new - ct)


def _prep_gate_weights(W_ih, b_ih, W_hh, b_hh):
    def reorder(w):
        g4 = w.reshape(4, H, -1)
        return jnp.concatenate(
            [0.5 * g4[0], 0.5 * g4[1], 0.5 * g4[3], g4[2]], axis=0)

    wih = reorder(W_ih)
    whh = reorder(W_hh)
    b = reorder((b_ih + b_hh)[:, None])
    return wih, whh, b


def kernel(corr_index, rela_ht, rela_ct, nei_index, W_emb, b_emb, W_ih, b_ih,
           W_hh, b_hh):
    htT = rela_ht.transpose(0, 2, 1)
    ctT = rela_ct.transpose(0, 2, 1)
    aux = jnp.concatenate([
        corr_index.transpose(2, 0, 1),
        nei_index.astype(jnp.float32)[None, :, :],
    ], axis=0)
    wfront = jnp.zeros((EMB + H, 3), dtype=jnp.float32)
    wfront = wfront.at[:EMB, 0:2].set(W_emb)
    wfront = wfront.at[EMB:, 2].set(1.0)
    bfront = jnp.concatenate([b_emb, jnp.zeros((H,), jnp.float32)])
    bfront = bfront.reshape(EMB + H, 1)
    wih, whh, b = _prep_gate_weights(W_ih, b_ih, W_hh, b_hh)

    ht_out, ct_out = pl.pallas_call(
        _cell_kernel,
        grid=(P // B,),
        in_specs=[
            pl.BlockSpec((3, B, P), lambda i: (0, i, 0)),
            pl.BlockSpec((B, H, P), lambda i: (i, 0, 0)),
            pl.BlockSpec((B, H, P), lambda i: (i, 0, 0)),
            pl.BlockSpec((EMB + H, 3), lambda i: (0, 0)),
            pl.BlockSpec((EMB + H, 1), lambda i: (0, 0)),
            pl.BlockSpec((4 * H, EMB), lambda i: (0, 0)),
            pl.BlockSpec((4 * H, H), lambda i: (0, 0)),
            pl.BlockSpec((4 * H, 1), lambda i: (0, 0)),
        ],
        out_specs=[
            pl.BlockSpec((B, H, P), lambda i: (i, 0, 0)),
            pl.BlockSpec((B, H, P), lambda i: (i, 0, 0)),
        ],
        out_shape=[jax.ShapeDtypeStruct((P, H, P), jnp.float32)] * 2,
    )(aux, htT, ctT, wfront, bfront, wih, whh, b)
    return ht_out.transpose(0, 2, 1), ct_out.transpose(0, 2, 1)


# speedup vs baseline: 1.1357x; 1.0102x over previous
import jax
import jax.numpy as jnp
from jax.experimental import pallas as pl

P = 512
EMB = 32
H = 64
B = 32


def _cell_kernel(aux_ref, ht_ref, ct_ref, wfront_ref, bfront_ref,
                 wih_ref, whh_ref, b_ref, hout_ref, cout_ref):
    wfront = wfront_ref[...]
    bfront = bfront_ref[...]
    wih = wih_ref[...]
    whh = whh_ref[...]
    b = b_ref[...]
    for k in range(B):
        ht = ht_ref[k]
        ct = ct_ref[k]
        front = jnp.maximum(
            jnp.dot(wfront, aux_ref[:, k, :],
                    preferred_element_type=jnp.float32) + bfront,
            0.0)
        emb = front[:EMB, :]
        m = front[EMB:, :]
        gates = (jnp.dot(wih, emb, preferred_element_type=jnp.float32) +
                 jnp.dot(whh, ht, preferred_element_type=jnp.float32) + b)
        t = jnp.tanh(gates)
        sig = 0.5 + 0.5 * t[0:3 * H, :]
        i_g = sig[0 * H:1 * H, :]
        f_g = sig[1 * H:2 * H, :]
        o_g = sig[2 * H:3 * H, :]
        g_g = t[3 * H:4 * H, :]
        c_new = f_g * ct + i_g * g_g
        h_new = o_g * jnp.tanh(c_new)
        hout_ref[k] = ht + m * (h_new - ht)
        cout_ref[k] = ct + m * (c_new - ct)


def _prep_gate_weights(W_ih, b_ih, W_hh, b_hh):
    def reorder(w):
        g4 = w.reshape(4, H, -1)
        return jnp.concatenate(
            [0.5 * g4[0], 0.5 * g4[1], 0.5 * g4[3], g4[2]], axis=0)

    wih = reorder(W_ih)
    whh = reorder(W_hh)
    b = reorder((b_ih + b_hh)[:, None])
    return wih, whh, b


def kernel(corr_index, rela_ht, rela_ct, nei_index, W_emb, b_emb, W_ih, b_ih,
           W_hh, b_hh):
    htT = rela_ht.transpose(0, 2, 1)
    ctT = rela_ct.transpose(0, 2, 1)
    aux = jnp.concatenate([
        corr_index.transpose(2, 0, 1),
        nei_index.astype(jnp.float32)[None, :, :],
    ], axis=0)
    wfront = jnp.zeros((EMB + H, 3), dtype=jnp.float32)
    wfront = wfront.at[:EMB, 0:2].set(W_emb)
    wfront = wfront.at[EMB:, 2].set(1.0)
    bfront = jnp.concatenate([b_emb, jnp.zeros((H,), jnp.float32)])
    bfront = bfront.reshape(EMB + H, 1)
    wih, whh, b = _prep_gate_weights(W_ih, b_ih, W_hh, b_hh)

    ht_out, ct_out = pl.pallas_call(
        _cell_kernel,
        grid=(P // B,),
        in_specs=[
            pl.BlockSpec((3, B, P), lambda i: (0, i, 0)),
            pl.BlockSpec((B, H, P), lambda i: (i, 0, 0)),
            pl.BlockSpec((B, H, P), lambda i: (i, 0, 0)),
            pl.BlockSpec((EMB + H, 3), lambda i: (0, 0)),
            pl.BlockSpec((EMB + H, 1), lambda i: (0, 0)),
            pl.BlockSpec((4 * H, EMB), lambda i: (0, 0)),
            pl.BlockSpec((4 * H, H), lambda i: (0, 0)),
            pl.BlockSpec((4 * H, 1), lambda i: (0, 0)),
        ],
        out_specs=[
            pl.BlockSpec((B, H, P), lambda i: (i, 0, 0)),
            pl.BlockSpec((B, H, P), lambda i: (i, 0, 0)),
        ],
        out_shape=[jax.ShapeDtypeStruct((P, H, P), jnp.float32)] * 2,
    )(aux, htT, ctT, wfront, bfront, wih, whh, b)
    return ht_out.transpose(0, 2, 1), ct_out.transpose(0, 2, 1)
